# baseline (speedup 1.0000x reference)
import jax
import jax.numpy as jnp
from jax import lax
from jax.experimental import pallas as pl
from jax.experimental.pallas import tpu as pltpu
from jax.experimental.pallas import tpu_sc as plsc

N = 10000
E = 320000
D = 128
R = 32

NC = 2
NS = 16
DH = D // NC
EPW = E // NS
CHUNK = 125
NCHUNK = EPW // CHUNK
NBUF = 4
NIB = 6
MROWS = 5
ROWS_PER_TILE = 624
ZROWS = 104
REM_ROWS = N - NS * ROWS_PER_TILE
DEG_PER_TILE = 1000

_F32 = jnp.float32


def _sc_mesh():
    return plsc.VectorSubcoreMesh(
        core_axis_name="c", subcore_axis_name="s", num_cores=NC, num_subcores=NS
    )


def _make_sc_kernel(compute_deg):
    out_type = [jax.ShapeDtypeStruct((NC, N, DH), _F32)]
    if compute_deg:
        out_type.append(jax.ShapeDtypeStruct((N,), _F32))

    scratch = [
        pltpu.VMEM((NIB, 3, CHUNK), jnp.int32),
        pltpu.VMEM((NBUF, CHUNK, DH), _F32),
        pltpu.VMEM((NBUF, CHUNK, DH), _F32),
        pltpu.VMEM((ZROWS, DH), _F32),
        pltpu.SemaphoreType.DMA,
        pltpu.SemaphoreType.DMA,
        pltpu.SemaphoreType.DMA,
        pltpu.SemaphoreType.DMA,
        pltpu.VMEM_SHARED((N, DH), _F32),
    ]
    if compute_deg:
        scratch += [
            pltpu.VMEM((128,), _F32),
            pltpu.VMEM((DEG_PER_TILE,), _F32),
            pltpu.SemaphoreType.DMA,
            pltpu.VMEM_SHARED((N,), _F32),
        ]

    def body(x3_hbm, rel3_hbm, eidx_hbm, *rest):
        if compute_deg:
            (aggh_hbm, degp_hbm, idx_v, xr_v, rr_v, zbuf, sem_gi, sem_gx,
             sem_gr, sem_sc, agg_sh, ones_v, zdeg, sem_dg, deg_sh) = rest
        else:
            (aggh_hbm, idx_v, xr_v, rr_v, zbuf, sem_gi, sem_gx,
             sem_gr, sem_sc, agg_sh) = rest

        cid = lax.axis_index("c")
        sid = lax.axis_index("s")
        x_tab = x3_hbm.at[cid]
        rel_tab = rel3_hbm.at[cid]

        zv = jnp.zeros((16,), _F32)

        def zrow(i, carry):
            for c4 in range(DH // 16):
                zbuf[i, pl.ds(c4 * 16, 16)] = zv
            return carry

        lax.fori_loop(0, ZROWS, zrow, 0)
        base = sid * ROWS_PER_TILE
        for k in range(ROWS_PER_TILE // ZROWS):
            pltpu.sync_copy(zbuf, agg_sh.at[pl.ds(base + k * ZROWS, ZROWS), :])

        @pl.when(sid == 0)
        def _():
            pltpu.sync_copy(
                zbuf.at[pl.ds(0, REM_ROWS), :],
                agg_sh.at[pl.ds(NS * ROWS_PER_TILE, REM_ROWS), :],
            )

        if compute_deg:
            ov = jnp.ones((16,), _F32)

            def fill1(i, carry):
                ones_v[pl.ds(i * 16, 16)] = ov
                return carry

            lax.fori_loop(0, 128 // 16, fill1, 0)

            def zd(i, carry):
                zdeg[pl.ds(i * 16, 16)] = zv
                return carry

            lax.fori_loop(0, DEG_PER_TILE // 16, zd, 0)

            @pl.when(sid < N // DEG_PER_TILE)
            def _():
                pltpu.sync_copy(
                    zdeg, deg_sh.at[pl.ds(sid * DEG_PER_TILE, DEG_PER_TILE)]
                )

        plsc.subcore_barrier()

        def idx_load(j):
            pltpu.async_copy(
                eidx_hbm.at[sid, j], idx_v.at[lax.rem(j, NIB)], sem_gi
            )

        def idx_wait(j):
            pltpu.make_async_copy(
                eidx_hbm.at[sid, j], idx_v.at[lax.rem(j, NIB)], sem_gi
            ).wait()

        def issue_gathers(j):
            b = lax.rem(j, NBUF)
            i = lax.rem(j, NIB)
            pltpu.async_copy(x_tab.at[idx_v.at[i, 0]], xr_v.at[b], sem_gx)
            pltpu.async_copy(rel_tab.at[idx_v.at[i, 2]], rr_v.at[b], sem_gr)

        def wait_scatter(j):
            pltpu.make_async_copy(
                xr_v.at[lax.rem(j, NBUF)],
                agg_sh.at[idx_v.at[lax.rem(j, NIB), 1]],
                sem_sc,
            ).wait()

        def wait_deg(j):
            pltpu.make_async_copy(
                ones_v.at[pl.ds(0, CHUNK)],
                deg_sh.at[idx_v.at[lax.rem(j, NIB), 1]],
                sem_dg,
            ).wait()

        for jj in range(NBUF + 1):
            idx_load(jj)
        for jj in range(NBUF - 1):
            idx_wait(jj)
            issue_gathers(jj)

        def step(j, carry):
            b = lax.rem(j, NBUF)

            @pl.when(j >= 1)
            def _():
                wait_scatter(j - 1)
                if compute_deg:
                    wait_deg(j - 1)

            @pl.when(j + NBUF + 1 < NCHUNK)
            def _():
                idx_load(j + NBUF + 1)

            @pl.when(j + NBUF - 1 < NCHUNK)
            def _():
                idx_wait(j + NBUF - 1)
                issue_gathers(j + NBUF - 1)

            pltpu.make_async_copy(
                x_tab.at[idx_v.at[lax.rem(j, NIB), 0]], xr_v.at[b], sem_gx
            ).wait()
            pltpu.make_async_copy(
                rel_tab.at[idx_v.at[lax.rem(j, NIB), 2]], rr_v.at[b], sem_gr
            ).wait()

            xc = xr_v.at[b]
            rc = rr_v.at[b]

            def mulrow(g, c):
                r0 = g * MROWS
                for dr in range(MROWS):
                    for c4 in range(DH // 16):
                        sl = pl.ds(c4 * 16, 16)
                        xc[r0 + dr, sl] = xc[r0 + dr, sl] * rc[r0 + dr, sl]
                return c

            lax.fori_loop(0, CHUNK // MROWS, mulrow, 0)

            pltpu.async_copy(
                xr_v.at[b], agg_sh.at[idx_v.at[lax.rem(j, NIB), 1]], sem_sc,
                add=True,
            )
            if compute_deg:
                pltpu.async_copy(
                    ones_v.at[pl.ds(0, CHUNK)],
                    deg_sh.at[idx_v.at[lax.rem(j, NIB), 1]],
                    sem_dg, add=True,
                )
            return carry

        lax.fori_loop(0, NCHUNK, step, 0)
        wait_scatter(NCHUNK - 1)
        if compute_deg:
            wait_deg(NCHUNK - 1)

        plsc.subcore_barrier()

        for k in range(ROWS_PER_TILE // ZROWS):
            sl = pl.ds(base + k * ZROWS, ZROWS)
            pltpu.sync_copy(agg_sh.at[sl, :], aggh_hbm.at[cid, sl, :])

        @pl.when(sid == 0)
        def _():
            sl = pl.ds(NS * ROWS_PER_TILE, REM_ROWS)
            pltpu.sync_copy(agg_sh.at[sl, :], aggh_hbm.at[cid, sl, :])

        if compute_deg:
            @pl.when((cid == 0) & (sid < N // DEG_PER_TILE))
            def _():
                pltpu.sync_copy(
                    deg_sh.at[pl.ds(sid * DEG_PER_TILE, DEG_PER_TILE)], zdeg
                )
                pltpu.sync_copy(
                    zdeg, degp_hbm.at[pl.ds(sid * DEG_PER_TILE, DEG_PER_TILE)]
                )

    return pl.kernel(
        body, out_type=out_type, mesh=_sc_mesh(), scratch_types=scratch,
        compiler_params=pltpu.CompilerParams(use_tc_tiling_on_sc=False),
    )


_sc_layer_with_deg = _make_sc_kernel(compute_deg=True)
_sc_layer_no_deg = _make_sc_kernel(compute_deg=False)

BN = 1000


def _make_tc_kernel(apply_act, emit_split):
    def body(aggh_ref, degc_ref, xa_ref, xb_ref, wmsg_ref, wself_ref,
             rela_ref, relb_ref, wrel_ref, xo_ref, ro_ref):
        i = pl.program_id(0)
        inv = 1.0 / jnp.maximum(degc_ref[...], 1.0)
        h = jnp.dot(aggh_ref[0] * inv, wmsg_ref[0:DH, :],
                    preferred_element_type=_F32)
        h = h + jnp.dot(aggh_ref[1] * inv, wmsg_ref[DH:D, :],
                        preferred_element_type=_F32)
        h = h + jnp.dot(xa_ref[...], wself_ref[0:DH, :],
                        preferred_element_type=_F32)
        h = h + jnp.dot(xb_ref[...], wself_ref[DH:D, :],
                        preferred_element_type=_F32)
        if apply_act:
            h = jnp.maximum(h, 0.0)
        if emit_split:
            xo_ref[0] = h[:, 0:DH]
            xo_ref[1] = h[:, DH:D]
        else:
            xo_ref[...] = h

        @pl.when(i == 0)
        def _():
            ro = jnp.dot(rela_ref[...], wrel_ref[0:DH, :],
                         preferred_element_type=_F32)
            ro = ro + jnp.dot(relb_ref[...], wrel_ref[DH:D, :],
                              preferred_element_type=_F32)
            if apply_act:
                ro = jnp.maximum(ro, 0.0)
            if emit_split:
                for k in range(NS):
                    ro_ref[0, k * R:(k + 1) * R, :] = ro[:, 0:DH]
                    ro_ref[1, k * R:(k + 1) * R, :] = ro[:, DH:D]
            else:
                ro_ref[...] = ro

    grid = (N // BN,)
    in_specs = [
        pl.BlockSpec((NC, BN, DH), lambda i: (0, i, 0)),
        pl.BlockSpec((BN, 1), lambda i: (i, 0)),
        pl.BlockSpec((BN, DH), lambda i: (i, 0)),
        pl.BlockSpec((BN, DH), lambda i: (i, 0)),
        pl.BlockSpec((D, D), lambda i: (0, 0)),
        pl.BlockSpec((D, D), lambda i: (0, 0)),
        pl.BlockSpec((R, DH), lambda i: (0, 0)),
        pl.BlockSpec((R, DH), lambda i: (0, 0)),
        pl.BlockSpec((D, D), lambda i: (0, 0)),
    ]
    if emit_split:
        out_specs = [
            pl.BlockSpec((NC, BN, DH), lambda i: (0, i, 0)),
            pl.BlockSpec((NC, NS * R, DH), lambda i: (0, 0, 0)),
        ]
        out_shape = [
            jax.ShapeDtypeStruct((NC, N, DH), _F32),
            jax.ShapeDtypeStruct((NC, NS * R, DH), _F32),
        ]
    else:
        out_specs = [
            pl.BlockSpec((BN, D), lambda i: (i, 0)),
            pl.BlockSpec((R, D), lambda i: (0, 0)),
        ]
        out_shape = [
            jax.ShapeDtypeStruct((N, D), _F32),
            jax.ShapeDtypeStruct((R, D), _F32),
        ]
    return pl.pallas_call(
        body, grid=grid, in_specs=in_specs, out_specs=out_specs,
        out_shape=out_shape,
    )


_tc_layer1 = _make_tc_kernel(apply_act=True, emit_split=True)
_tc_layer2 = _make_tc_kernel(apply_act=False, emit_split=False)


def kernel(x, edge_index, edge_type, rel_embed, W_msg1, W_self1, W_rel1,
           W_msg2, W_self2, W_rel2):
    src = edge_index[0].reshape(NS, NCHUNK, 1, CHUNK)
    dst = edge_index[1].reshape(NS, NCHUNK, 1, CHUNK)
    et = (edge_type.reshape(NS, NCHUNK, 1, CHUNK)
          + (jnp.arange(NS, dtype=jnp.int32) * R).reshape(NS, 1, 1, 1))
    eidx = jnp.concatenate([src, dst, et], axis=2)

    x3 = x.reshape(N, NC, DH).transpose(1, 0, 2)
    rel3 = rel_embed.reshape(R, NC, DH).transpose(1, 0, 2)
    rel3_rep = jnp.broadcast_to(
        rel3[:, None], (NC, NS, R, DH)).reshape(NC, NS * R, DH)

    aggh1, degp = _sc_layer_with_deg(x3, rel3_rep, eidx)
    degc = degp.reshape(N, 1)
    x1s, r1rep = _tc_layer1(aggh1, degc, x3[0], x3[1], W_msg1, W_self1,
                            rel3[0], rel3[1], W_rel1)
    (aggh2,) = _sc_layer_no_deg(x1s, r1rep, eidx)
    x2, r2 = _tc_layer2(aggh2, degc, x1s[0], x1s[1], W_msg2, W_self2,
                        r1rep[0, 0:R], r1rep[1, 0:R], W_rel2)
    return (x2, r2)


# speedup vs baseline: 2.0710x; 2.0710x over previous
import jax
import jax.numpy as jnp
from jax import lax
from jax.experimental import pallas as pl
from jax.experimental.pallas import tpu as pltpu
from jax.experimental.pallas import tpu_sc as plsc

N = 10000
E = 320000
D = 128
R = 32

NC = 2
NS = 16
DH = D // NC
EPW = E // NS
CHUNK = 125
NCHUNK = EPW // CHUNK
NBUF = 4
NIB = 6
MROWS = 5
ROWS_PER_TILE = 624
ZROWS = 104
REM_ROWS = N - NS * ROWS_PER_TILE
DEG_PER_TILE = 1000

_F32 = jnp.float32


def _sc_mesh():
    return plsc.VectorSubcoreMesh(
        core_axis_name="c", subcore_axis_name="s", num_cores=NC, num_subcores=NS
    )


def _make_sc_kernel(compute_deg):
    out_type = [jax.ShapeDtypeStruct((NC, N, DH), _F32)]
    if compute_deg:
        out_type.append(jax.ShapeDtypeStruct((N,), _F32))

    scratch = [
        pltpu.VMEM((NIB, 3, CHUNK), jnp.int32),
        pltpu.VMEM((NBUF, CHUNK, DH), _F32),
        pltpu.VMEM((NBUF, CHUNK, DH), _F32),
        pltpu.VMEM((ZROWS, DH), _F32),
        pltpu.SemaphoreType.DMA,
        pltpu.SemaphoreType.DMA,
        pltpu.SemaphoreType.DMA,
        pltpu.SemaphoreType.DMA,
        pltpu.VMEM_SHARED((N, DH), _F32),
    ]
    if compute_deg:
        scratch += [
            pltpu.VMEM((128,), _F32),
            pltpu.VMEM((DEG_PER_TILE,), _F32),
            pltpu.SemaphoreType.DMA,
            pltpu.VMEM_SHARED((N,), _F32),
        ]

    def body(x3_hbm, rel3_hbm, eidx_hbm, *rest):
        if compute_deg:
            (aggh_hbm, degp_hbm, idx_v, xr_v, rr_v, zbuf, sem_gi, sem_gx,
             sem_gr, sem_sc, agg_sh, ones_v, zdeg, sem_dg, deg_sh) = rest
        else:
            (aggh_hbm, idx_v, xr_v, rr_v, zbuf, sem_gi, sem_gx,
             sem_gr, sem_sc, agg_sh) = rest

        cid = lax.axis_index("c")
        sid = lax.axis_index("s")
        x_tab = x3_hbm.at[cid]
        rel_tab = rel3_hbm.at[cid]

        zv = jnp.zeros((16,), _F32)

        def zrow(i, carry):
            for c4 in range(DH // 16):
                zbuf[i, pl.ds(c4 * 16, 16)] = zv
            return carry

        lax.fori_loop(0, ZROWS, zrow, 0)
        base = sid * ROWS_PER_TILE
        for k in range(ROWS_PER_TILE // ZROWS):
            pltpu.sync_copy(zbuf, agg_sh.at[pl.ds(base + k * ZROWS, ZROWS), :])

        @pl.when(sid == 0)
        def _():
            pltpu.sync_copy(
                zbuf.at[pl.ds(0, REM_ROWS), :],
                agg_sh.at[pl.ds(NS * ROWS_PER_TILE, REM_ROWS), :],
            )

        if compute_deg:
            ov = jnp.ones((16,), _F32)

            def fill1(i, carry):
                ones_v[pl.ds(i * 16, 16)] = ov
                return carry

            lax.fori_loop(0, 128 // 16, fill1, 0)

            def zd(i, carry):
                zdeg[pl.ds(i * 16, 16)] = zv
                return carry

            lax.fori_loop(0, DEG_PER_TILE // 16, zd, 0)

            @pl.when(sid < N // DEG_PER_TILE)
            def _():
                pltpu.sync_copy(
                    zdeg, deg_sh.at[pl.ds(sid * DEG_PER_TILE, DEG_PER_TILE)]
                )

        plsc.subcore_barrier()

        def idx_load(j):
            pltpu.async_copy(
                eidx_hbm.at[sid, j], idx_v.at[lax.rem(j, NIB)], sem_gi
            )

        def idx_wait(j):
            pltpu.make_async_copy(
                eidx_hbm.at[sid, j], idx_v.at[lax.rem(j, NIB)], sem_gi
            ).wait()

        def issue_gathers(j):
            b = lax.rem(j, NBUF)
            i = lax.rem(j, NIB)
            pltpu.async_copy(x_tab.at[idx_v.at[i, 0]], xr_v.at[b], sem_gx)
            pltpu.async_copy(rel_tab.at[idx_v.at[i, 2]], rr_v.at[b], sem_gr)

        def wait_scatter(j):
            pltpu.make_async_copy(
                xr_v.at[lax.rem(j, NBUF)],
                agg_sh.at[idx_v.at[lax.rem(j, NIB), 1]],
                sem_sc,
            ).wait()

        def wait_deg(j):
            pltpu.make_async_copy(
                ones_v.at[pl.ds(0, CHUNK)],
                deg_sh.at[idx_v.at[lax.rem(j, NIB), 1]],
                sem_dg,
            ).wait()

        for jj in range(NBUF + 1):
            idx_load(jj)
        for jj in range(NBUF - 1):
            idx_wait(jj)
            issue_gathers(jj)

        def step(j, carry):
            b = lax.rem(j, NBUF)

            @pl.when(j >= 1)
            def _():
                wait_scatter(j - 1)
                if compute_deg:
                    wait_deg(j - 1)

            @pl.when(j + NBUF + 1 < NCHUNK)
            def _():
                idx_load(j + NBUF + 1)

            @pl.when(j + NBUF - 1 < NCHUNK)
            def _():
                idx_wait(j + NBUF - 1)
                issue_gathers(j + NBUF - 1)

            pltpu.make_async_copy(
                x_tab.at[idx_v.at[lax.rem(j, NIB), 0]], xr_v.at[b], sem_gx
            ).wait()
            pltpu.make_async_copy(
                rel_tab.at[idx_v.at[lax.rem(j, NIB), 2]], rr_v.at[b], sem_gr
            ).wait()

            xc = xr_v.at[b]
            rc = rr_v.at[b]

            @plsc.parallel_loop(0, CHUNK, step=1, unroll=MROWS)
            def mulrow(r):
                for c4 in range(DH // 16):
                    sl = pl.ds(c4 * 16, 16)
                    xc[r, sl] = xc[r, sl] * rc[r, sl]

            pltpu.async_copy(
                xr_v.at[b], agg_sh.at[idx_v.at[lax.rem(j, NIB), 1]], sem_sc,
                add=True,
            )
            if compute_deg:
                pltpu.async_copy(
                    ones_v.at[pl.ds(0, CHUNK)],
                    deg_sh.at[idx_v.at[lax.rem(j, NIB), 1]],
                    sem_dg, add=True,
                )
            return carry

        lax.fori_loop(0, NCHUNK, step, 0)
        wait_scatter(NCHUNK - 1)
        if compute_deg:
            wait_deg(NCHUNK - 1)

        plsc.subcore_barrier()

        for k in range(ROWS_PER_TILE // ZROWS):
            sl = pl.ds(base + k * ZROWS, ZROWS)
            pltpu.sync_copy(agg_sh.at[sl, :], aggh_hbm.at[cid, sl, :])

        @pl.when(sid == 0)
        def _():
            sl = pl.ds(NS * ROWS_PER_TILE, REM_ROWS)
            pltpu.sync_copy(agg_sh.at[sl, :], aggh_hbm.at[cid, sl, :])

        if compute_deg:
            @pl.when((cid == 0) & (sid < N // DEG_PER_TILE))
            def _():
                pltpu.sync_copy(
                    deg_sh.at[pl.ds(sid * DEG_PER_TILE, DEG_PER_TILE)], zdeg
                )
                pltpu.sync_copy(
                    zdeg, degp_hbm.at[pl.ds(sid * DEG_PER_TILE, DEG_PER_TILE)]
                )

    return pl.kernel(
        body, out_type=out_type, mesh=_sc_mesh(), scratch_types=scratch,
        compiler_params=pltpu.CompilerParams(use_tc_tiling_on_sc=False),
    )


_sc_layer_with_deg = _make_sc_kernel(compute_deg=True)
_sc_layer_no_deg = _make_sc_kernel(compute_deg=False)

BN = 1000


def _make_tc_kernel(apply_act, emit_split):
    def body(aggh_ref, degc_ref, xa_ref, xb_ref, wmsg_ref, wself_ref,
             rela_ref, relb_ref, wrel_ref, xo_ref, ro_ref):
        i = pl.program_id(0)
        inv = 1.0 / jnp.maximum(degc_ref[...], 1.0)
        h = jnp.dot(aggh_ref[0] * inv, wmsg_ref[0:DH, :],
                    preferred_element_type=_F32)
        h = h + jnp.dot(aggh_ref[1] * inv, wmsg_ref[DH:D, :],
                        preferred_element_type=_F32)
        h = h + jnp.dot(xa_ref[...], wself_ref[0:DH, :],
                        preferred_element_type=_F32)
        h = h + jnp.dot(xb_ref[...], wself_ref[DH:D, :],
                        preferred_element_type=_F32)
        if apply_act:
            h = jnp.maximum(h, 0.0)
        if emit_split:
            xo_ref[0] = h[:, 0:DH]
            xo_ref[1] = h[:, DH:D]
        else:
            xo_ref[...] = h

        @pl.when(i == 0)
        def _():
            ro = jnp.dot(rela_ref[...], wrel_ref[0:DH, :],
                         preferred_element_type=_F32)
            ro = ro + jnp.dot(relb_ref[...], wrel_ref[DH:D, :],
                              preferred_element_type=_F32)
            if apply_act:
                ro = jnp.maximum(ro, 0.0)
            if emit_split:
                for k in range(NS):
                    ro_ref[0, k * R:(k + 1) * R, :] = ro[:, 0:DH]
                    ro_ref[1, k * R:(k + 1) * R, :] = ro[:, DH:D]
            else:
                ro_ref[...] = ro

    grid = (N // BN,)
    in_specs = [
        pl.BlockSpec((NC, BN, DH), lambda i: (0, i, 0)),
        pl.BlockSpec((BN, 1), lambda i: (i, 0)),
        pl.BlockSpec((BN, DH), lambda i: (i, 0)),
        pl.BlockSpec((BN, DH), lambda i: (i, 0)),
        pl.BlockSpec((D, D), lambda i: (0, 0)),
        pl.BlockSpec((D, D), lambda i: (0, 0)),
        pl.BlockSpec((R, DH), lambda i: (0, 0)),
        pl.BlockSpec((R, DH), lambda i: (0, 0)),
        pl.BlockSpec((D, D), lambda i: (0, 0)),
    ]
    if emit_split:
        out_specs = [
            pl.BlockSpec((NC, BN, DH), lambda i: (0, i, 0)),
            pl.BlockSpec((NC, NS * R, DH), lambda i: (0, 0, 0)),
        ]
        out_shape = [
            jax.ShapeDtypeStruct((NC, N, DH), _F32),
            jax.ShapeDtypeStruct((NC, NS * R, DH), _F32),
        ]
    else:
        out_specs = [
            pl.BlockSpec((BN, D), lambda i: (i, 0)),
            pl.BlockSpec((R, D), lambda i: (0, 0)),
        ]
        out_shape = [
            jax.ShapeDtypeStruct((N, D), _F32),
            jax.ShapeDtypeStruct((R, D), _F32),
        ]
    return pl.pallas_call(
        body, grid=grid, in_specs=in_specs, out_specs=out_specs,
        out_shape=out_shape,
    )


_tc_layer1 = _make_tc_kernel(apply_act=True, emit_split=True)
_tc_layer2 = _make_tc_kernel(apply_act=False, emit_split=False)


def kernel(x, edge_index, edge_type, rel_embed, W_msg1, W_self1, W_rel1,
           W_msg2, W_self2, W_rel2):
    src = edge_index[0].reshape(NS, NCHUNK, 1, CHUNK)
    dst = edge_index[1].reshape(NS, NCHUNK, 1, CHUNK)
    et = (edge_type.reshape(NS, NCHUNK, 1, CHUNK)
          + (jnp.arange(NS, dtype=jnp.int32) * R).reshape(NS, 1, 1, 1))
    eidx = jnp.concatenate([src, dst, et], axis=2)

    x3 = x.reshape(N, NC, DH).transpose(1, 0, 2)
    rel3 = rel_embed.reshape(R, NC, DH).transpose(1, 0, 2)
    rel3_rep = jnp.broadcast_to(
        rel3[:, None], (NC, NS, R, DH)).reshape(NC, NS * R, DH)

    aggh1, degp = _sc_layer_with_deg(x3, rel3_rep, eidx)
    degc = degp.reshape(N, 1)
    x1s, r1rep = _tc_layer1(aggh1, degc, x3[0], x3[1], W_msg1, W_self1,
                            rel3[0], rel3[1], W_rel1)
    (aggh2,) = _sc_layer_no_deg(x1s, r1rep, eidx)
    x2, r2 = _tc_layer2(aggh2, degc, x1s[0], x1s[1], W_msg2, W_self2,
                        r1rep[0, 0:R], r1rep[1, 0:R], W_rel2)
    return (x2, r2)
